# baseline (speedup 1.0000x reference)
import functools

import jax
import jax.numpy as jnp
from jax.experimental import pallas as pl
from jax.experimental.pallas import tpu as pltpu

C = 128


def _dot(a, b):
    return jnp.dot(a, b, preferred_element_type=jnp.float32)


def _dot_t(a, b):
    return jax.lax.dot_general(a, b, (((1,), (1,)), ((), ())),
                               preferred_element_type=jnp.float32)


def _dot_tt(a, b):
    return jax.lax.dot_general(a, b, (((0,), (0,)), ((), ())),
                               preferred_element_type=jnp.float32)


def _proj_kernel(x_ref, wqkv_ref, ww1_ref, ww2_ref, wb_ref, wf_ref, delta_ref,
                 q_ref, k_ref, v_ref, w_ref, beta_ref, logf_ref, *, h, hd):
    xb = x_ref[0]
    qkv = _dot(xb, wqkv_ref[...])
    xw1 = _dot(xb, ww1_ref[...])
    wfull = _dot(xw1, ww2_ref[...])
    d = xb.shape[1]
    blog = _dot(xb, wb_ref[...])
    flog = _dot(xb, wf_ref[...]) + delta_ref[...]
    beta = 2.0 * jax.nn.sigmoid(blog)
    logf = jax.nn.log_sigmoid(flog)
    beta_ref[0, :, 0, :] = jnp.transpose(beta)
    logf_ref[0, :, 0, :] = jnp.transpose(logf)
    for hh in range(h):
        sl = slice(hh * hd, (hh + 1) * hd)
        q_ref[hh] = qkv[:, sl]
        k_ref[hh] = qkv[:, d + hh * hd:d + (hh + 1) * hd]
        ones = jnp.ones((qkv.shape[0], 1), jnp.float32)
        v_ref[hh] = jnp.concatenate(
            [qkv[:, 2 * d + hh * hd:2 * d + (hh + 1) * hd], ones], axis=1)
        wh = wfull[:, sl]
        w_ref[hh] = wh * jax.lax.rsqrt(
            jnp.sum(wh * wh, axis=1, keepdims=True) + 1e-6)


def _ut_kernel(q_ref, k_ref, w_ref, b_ref,
               qt_ref, kt_ref, sii_ref, p_ref, *, c, hd, newton_iters, nh,
               scale):
    rows = jax.lax.broadcasted_iota(jnp.int32, (c, c), 0)
    cols = jax.lax.broadcasted_iota(jnp.int32, (c, c), 1)
    lower = rows > cols
    lower_eq = rows >= cols
    eye = jnp.where(rows == cols, 1.0, 0.0).astype(jnp.float32)
    hrows = jax.lax.broadcasted_iota(jnp.int32, (hd, hd), 0)
    hcols = jax.lax.broadcasted_iota(jnp.int32, (hd, hd), 1)
    eye_h = jnp.where(hrows == hcols, 1.0, 0.0).astype(jnp.float32)

    Ws = [w_ref[t] for t in range(nh)]
    Ks = [k_ref[t] for t in range(nh)]
    Qs = [q_ref[t] for t in range(nh)]
    bvs = [b_ref[0, t] for t in range(nh)]
    Ls = [jnp.where(lower, _dot_t(W, W), 0.0) * bv
          for W, bv in zip(Ws, bvs)]
    h2 = c // 2
    it2 = max(1, (h2 - 1).bit_length() - 1)
    eye2 = eye[:h2, :h2]
    Lhs = ([L[:h2, :h2] for L in Ls] + [L[h2:, h2:] for L in Ls])
    M1h = [eye2 + L for L in Lhs]
    Xh = [eye2 - L for L in Lhs]
    for _ in range(it2):
        Yh = [_dot(M1, X) for M1, X in zip(M1h, Xh)]
        Xh = [_dot(X, 2.0 * eye2 - Y) for X, Y in zip(Xh, Yh)]
    T21s = [_dot(Ls[t][h2:, :h2], Xh[t]) for t in range(nh)]
    X21s = [-_dot(Xh[nh + t], T21s[t]) for t in range(nh)]
    z2 = jnp.zeros((h2, h2), jnp.float32)
    Xs = [jnp.concatenate(
        [jnp.concatenate([Xh[t], z2], axis=1),
         jnp.concatenate([X21s[t], Xh[nh + t]], axis=1)], axis=0)
        for t in range(nh)]
    Mlows = [jnp.where(lower, _dot_t(W, K), 0.0) for W, K in zip(Ws, Ks)]
    Gs = [jnp.where(lower_eq, _dot_t(Q, W), 0.0) * bv
          for Q, W, bv in zip(Qs, Ws, bvs)]
    GTs = [_dot(G, X) for G, X in zip(Gs, Xs)]
    Cms = [_dot(X, Mlow) for X, Mlow in zip(Xs, Mlows)]
    QKs = [_dot_t(Q, K) for Q, K in zip(Qs, Ks)]
    BWs = [jnp.transpose(bv) * W for bv, W in zip(bvs, Ws)]
    XWs = [_dot(X, W) for X, W in zip(Xs, Ws)]
    for t in range(nh):
        sii_ref[t, 0] = (QKs[t] - _dot(GTs[t], Mlows[t])) * scale
        qt_ref[t] = (Qs[t] - _dot(GTs[t], Ws[t])) * scale
        kt_ref[t] = Ks[t] - _dot_tt(Cms[t], BWs[t])
        p_ref[t, 0] = eye_h - _dot_tt(BWs[t], XWs[t])


def _flash_kernel(*refs, c, hd, nc, nb, clo, ncw, has_kin, has_kout):
    qt_ref, sii_ref, kt_ref, v_ref, pm_ref, logf_ref = refs[:6]
    i = 6
    kin_ref = None
    if has_kin:
        kin_ref = refs[i]
        i += 1
    o_ref = refs[i]
    i += 1
    kout_ref = None
    if has_kout:
        kout_ref = refs[i]
        i += 1
    fc_ref, khat_ref = refs[i], refs[i + 1]

    nw = clo + ncw
    ii_loc = pl.program_id(1)
    ii = ii_loc + clo

    @pl.when(ii_loc == 0)
    def _():
        khat_ref[...] = jnp.zeros_like(khat_ref)
        if has_kin:
            for t in range(nb):
                khat_ref[t, :clo * c, :] = kin_ref[t]
        crows = jax.lax.broadcasted_iota(jnp.int32, (c, c), 0)
        ccols = jax.lax.broadcasted_iota(jnp.int32, (c, c), 1)
        u_inc = jnp.where(crows <= ccols, 1.0, 0.0).astype(jnp.float32)
        nrows = jax.lax.broadcasted_iota(jnp.int32, (nc, nc), 0)
        ncols = jax.lax.broadcasted_iota(jnp.int32, (nc, nc), 1)
        l_strict = jnp.where(nrows > ncols, 1.0, 0.0).astype(jnp.float32)
        for t in range(nb):
            row_cum = _dot(logf_ref[0, t], u_inc)
            offs = _dot(l_strict, row_cum[:, c - 1:c])
            fc_ref[t] = row_cum + offs

    rows = jax.lax.broadcasted_iota(jnp.int32, (c, c), 0)
    cols = jax.lax.broadcasted_iota(jnp.int32, (c, c), 1)
    tri = rows >= cols

    fq_rows = [fc_ref[t, pl.ds(ii, 1), :] for t in range(nb)]
    fq_cols = [jnp.transpose(fq) for fq in fq_rows]
    Ss = [_dot_t(qt_ref[t], khat_ref[t]) for t in range(nb)]
    lbds = [jnp.where(tri, sii_ref[t, 0] + fq_cols[t] - fq_rows[t], -1e30)
            for t in range(nb)]
    ms = [lbd.max(axis=1, keepdims=True) for lbd in lbds]
    all_blocks = []
    for t in range(nb):
        blocks = []
        m = ms[t]
        for r in range(nw):
            lb = Ss[t][:, r * c:(r + 1) * c] + fq_cols[t] - fc_ref[t, r:r + 1, :]
            blocks.append(lb)
            mr = lb.max(axis=1, keepdims=True)
            m = jnp.maximum(m, jnp.where(r < ii, mr, -jnp.inf))
        ms[t] = m
        all_blocks.append(blocks)
    Eds = [jnp.exp(lbds[t] - ms[t]) for t in range(nb)]
    Es = [jnp.concatenate(
        [jnp.exp(bk - ms[t]) * jnp.where(r < ii, 1.0, 0.0)
         for r, bk in enumerate(all_blocks[t])], axis=1) for t in range(nb)]
    v_iis = [v_ref[t, pl.ds(ii * c, c), :] for t in range(nb)]
    pvls = [_dot(Es[t], v_ref[t]) + _dot(Eds[t], v_iis[t]) for t in range(nb)]
    for t in range(nb):
        o_ref[t] = pvls[t][:, :hd] / pvls[t][:, hd:hd + 1]
    for t in range(nb):
        khat_ref[t] = _dot_t(khat_ref[t], pm_ref[t, 0])
    for t in range(nb):
        khat_ref[t, pl.ds(ii * c, c), :] = kt_ref[t]
    if has_kout:
        @pl.when(ii_loc == ncw - 1)
        def _():
            for t in range(nb):
                kout_ref[t] = khat_ref[t]


def _outproj_kernel(o_ref, wo_ref, out_ref, *, h):
    om = jnp.concatenate([o_ref[hh] for hh in range(h)], axis=1)
    out_ref[0] = _dot(om, wo_ref[...])


def kernel(x, Wq, Wk, Wv, Wo, Ww1, Ww2, Wb, Wf, delta):
    b, n, d = x.shape
    h = delta.shape[0]
    hd = d // h
    nc = n // C
    bh = b * h
    scale = hd ** -0.5
    newton_iters = max(1, (C - 1).bit_length() - 1)

    wqkv = jnp.concatenate([Wq, Wk, Wv], axis=1)
    delta2 = delta.reshape(1, h)

    f32 = jnp.float32
    q, k, v, w, beta, logf = pl.pallas_call(
        functools.partial(_proj_kernel, h=h, hd=hd),
        grid=(b, nc),
        in_specs=[
            pl.BlockSpec((1, C, d), lambda bi, ci: (bi, ci, 0)),
            pl.BlockSpec((d, 3 * d), lambda bi, ci: (0, 0)),
            pl.BlockSpec((d, hd), lambda bi, ci: (0, 0)),
            pl.BlockSpec((hd, d), lambda bi, ci: (0, 0)),
            pl.BlockSpec((d, h), lambda bi, ci: (0, 0)),
            pl.BlockSpec((d, h), lambda bi, ci: (0, 0)),
            pl.BlockSpec((1, h), lambda bi, ci: (0, 0)),
        ],
        out_specs=[
            pl.BlockSpec((h, C, hd), lambda bi, ci: (bi, ci, 0)),
            pl.BlockSpec((h, C, hd), lambda bi, ci: (bi, ci, 0)),
            pl.BlockSpec((h, C, hd + 1), lambda bi, ci: (bi, ci, 0)),
            pl.BlockSpec((h, C, hd), lambda bi, ci: (bi, ci, 0)),
            pl.BlockSpec((1, h, 1, C), lambda bi, ci: (bi, 0, 0, ci)),
            pl.BlockSpec((1, h, 1, C), lambda bi, ci: (bi, 0, 0, ci)),
        ],
        out_shape=[
            jax.ShapeDtypeStruct((bh, n, hd), f32),
            jax.ShapeDtypeStruct((bh, n, hd), f32),
            jax.ShapeDtypeStruct((bh, n, hd + 1), f32),
            jax.ShapeDtypeStruct((bh, n, hd), f32),
            jax.ShapeDtypeStruct((b, h, 1, n), f32),
            jax.ShapeDtypeStruct((b, h, 1, n), f32),
        ],
        compiler_params=pltpu.CompilerParams(
            dimension_semantics=("parallel", "arbitrary"),
            vmem_limit_bytes=52 * 1024 * 1024,
        ),
        name="path_proj",
    )(x, wqkv, Ww1, Ww2, Wb, Wf, delta2)

    NH = 8
    qt, kt, sii, pm = pl.pallas_call(
        functools.partial(_ut_kernel, c=C, hd=hd, newton_iters=newton_iters,
                          nh=NH, scale=scale),
        grid=(bh // NH, nc),
        in_specs=[
            pl.BlockSpec((NH, C, hd), lambda g, ci: (g, ci, 0)),
            pl.BlockSpec((NH, C, hd), lambda g, ci: (g, ci, 0)),
            pl.BlockSpec((NH, C, hd), lambda g, ci: (g, ci, 0)),
            pl.BlockSpec((1, NH, 1, C),
                         lambda g, ci: (g // (h // NH), g % (h // NH), 0, ci)),
        ],
        out_specs=[
            pl.BlockSpec((NH, C, hd), lambda g, ci: (g, ci, 0)),
            pl.BlockSpec((NH, C, hd), lambda g, ci: (g, ci, 0)),
            pl.BlockSpec((NH, 1, C, C), lambda g, ci: (g, ci, 0, 0)),
            pl.BlockSpec((NH, 1, hd, hd), lambda g, ci: (g, ci, 0, 0)),
        ],
        out_shape=[
            jax.ShapeDtypeStruct((bh, n, hd), f32),
            jax.ShapeDtypeStruct((bh, n, hd), f32),
            jax.ShapeDtypeStruct((bh, nc, C, C), f32),
            jax.ShapeDtypeStruct((bh, nc, hd, hd), f32),
        ],
        compiler_params=pltpu.CompilerParams(
            dimension_semantics=("parallel", "arbitrary"),
        ),
        name="path_ut",
    )(q, k, w, beta)

    NB = 16
    ns = nc // 2
    o_parts = []
    khat_mid = None
    for clo, ncw in ((0, ns), (ns, nc - ns)):
        nw = clo + ncw
        has_kin = clo > 0
        has_kout = clo == 0
        in_specs = [
            pl.BlockSpec((NB, C, hd), lambda g, ii, clo=clo: (g, clo + ii, 0)),
            pl.BlockSpec((NB, 1, C, C),
                         lambda g, ii, clo=clo: (g, clo + ii, 0, 0)),
            pl.BlockSpec((NB, C, hd), lambda g, ii, clo=clo: (g, clo + ii, 0)),
            pl.BlockSpec((NB, nw * C, hd + 1), lambda g, ii: (g, 0, 0)),
            pl.BlockSpec((NB, 1, hd, hd),
                         lambda g, ii, clo=clo: (g, clo + ii, 0, 0)),
            pl.BlockSpec((1, NB, nc, C),
                         lambda g, ii: (g // (h // NB), g % (h // NB), 0, 0)),
        ]
        ins = [qt, sii, kt, v, pm, logf.reshape(b, h, nc, C)]
        if has_kin:
            in_specs.append(
                pl.BlockSpec((NB, clo * C, hd), lambda g, ii: (g, 0, 0)))
            ins.append(khat_mid)
        out_specs = [pl.BlockSpec((NB, C, hd), lambda g, ii: (g, ii, 0))]
        out_shape = [jax.ShapeDtypeStruct((bh, ncw * C, hd), f32)]
        if has_kout:
            out_specs.append(
                pl.BlockSpec((NB, nw * C, hd), lambda g, ii: (g, 0, 0)))
            out_shape.append(jax.ShapeDtypeStruct((bh, nw * C, hd), f32))
        res = pl.pallas_call(
            functools.partial(_flash_kernel, c=C, hd=hd, nc=nc, nb=NB,
                              clo=clo, ncw=ncw, has_kin=has_kin,
                              has_kout=has_kout),
            grid=(bh // NB, ncw),
            in_specs=in_specs,
            out_specs=out_specs,
            out_shape=out_shape,
            scratch_shapes=[
                pltpu.VMEM((NB, nc, C), f32),
                pltpu.VMEM((NB, nw * C, hd), f32),
            ],
            compiler_params=pltpu.CompilerParams(
                dimension_semantics=("parallel", "arbitrary"),
            ),
            name="path_flash",
        )(*ins)
        o_parts.append(res[0])
        if has_kout:
            khat_mid = res[1]
    o = jnp.concatenate(o_parts, axis=1)

    RB = 256
    out = pl.pallas_call(
        functools.partial(_outproj_kernel, h=h),
        grid=(b, n // RB),
        in_specs=[
            pl.BlockSpec((h, RB, hd), lambda bi, ri: (bi, ri, 0)),
            pl.BlockSpec((d, d), lambda bi, ri: (0, 0)),
        ],
        out_specs=pl.BlockSpec((1, RB, d), lambda bi, ri: (bi, ri, 0)),
        out_shape=jax.ShapeDtypeStruct((b, n, d), f32),
        compiler_params=pltpu.CompilerParams(
            dimension_semantics=("parallel", "arbitrary"),
        ),
        name="path_outproj",
    )(o, Wo)
    return out


# speedup vs baseline: 1.0238x; 1.0238x over previous
import functools

import jax
import jax.numpy as jnp
from jax.experimental import pallas as pl
from jax.experimental.pallas import tpu as pltpu

C = 128


def _dot(a, b):
    return jnp.dot(a, b, preferred_element_type=jnp.float32)


def _dot_t(a, b):
    return jax.lax.dot_general(a, b, (((1,), (1,)), ((), ())),
                               preferred_element_type=jnp.float32)


def _dot_tt(a, b):
    return jax.lax.dot_general(a, b, (((0,), (0,)), ((), ())),
                               preferred_element_type=jnp.float32)


def _proj_kernel(x_ref, wqkv_ref, ww1_ref, ww2_ref, wb_ref, wf_ref, delta_ref,
                 q_ref, k_ref, v_ref, w_ref, beta_ref, logf_ref, *, h, hd):
    xb = x_ref[0]
    qkv = _dot(xb, wqkv_ref[...])
    xw1 = _dot(xb, ww1_ref[...])
    wfull = _dot(xw1, ww2_ref[...])
    d = xb.shape[1]
    blog = _dot(xb, wb_ref[...])
    flog = _dot(xb, wf_ref[...]) + delta_ref[...]
    beta = 2.0 * jax.nn.sigmoid(blog)
    logf = jax.nn.log_sigmoid(flog)
    beta_ref[0, :, 0, :] = jnp.transpose(beta)
    logf_ref[0, :, 0, :] = jnp.transpose(logf)
    for hh in range(h):
        sl = slice(hh * hd, (hh + 1) * hd)
        q_ref[hh] = qkv[:, sl]
        k_ref[hh] = qkv[:, d + hh * hd:d + (hh + 1) * hd]
        ones = jnp.ones((qkv.shape[0], 1), jnp.float32)
        v_ref[hh] = jnp.concatenate(
            [qkv[:, 2 * d + hh * hd:2 * d + (hh + 1) * hd], ones], axis=1)
        wh = wfull[:, sl]
        w_ref[hh] = wh * jax.lax.rsqrt(
            jnp.sum(wh * wh, axis=1, keepdims=True) + 1e-6)


def _ut_kernel(q_ref, k_ref, w_ref, b_ref,
               qt_ref, kt_ref, sii_ref, p_ref, *, c, hd, newton_iters, nh,
               scale):
    rows = jax.lax.broadcasted_iota(jnp.int32, (c, c), 0)
    cols = jax.lax.broadcasted_iota(jnp.int32, (c, c), 1)
    lower = rows > cols
    lower_eq = rows >= cols
    eye = jnp.where(rows == cols, 1.0, 0.0).astype(jnp.float32)
    hrows = jax.lax.broadcasted_iota(jnp.int32, (hd, hd), 0)
    hcols = jax.lax.broadcasted_iota(jnp.int32, (hd, hd), 1)
    eye_h = jnp.where(hrows == hcols, 1.0, 0.0).astype(jnp.float32)

    Ws = [w_ref[t] for t in range(nh)]
    Ks = [k_ref[t] for t in range(nh)]
    Qs = [q_ref[t] for t in range(nh)]
    bvs = [b_ref[0, t] for t in range(nh)]
    Ls = [jnp.where(lower, _dot_t(W, W), 0.0) * bv
          for W, bv in zip(Ws, bvs)]
    h2 = c // 2
    it2 = max(1, (h2 - 1).bit_length() - 1)
    eye2 = eye[:h2, :h2]
    Lhs = ([L[:h2, :h2] for L in Ls] + [L[h2:, h2:] for L in Ls])
    M1h = [eye2 + L for L in Lhs]
    Xh = [eye2 - L for L in Lhs]
    for _ in range(it2):
        Yh = [_dot(M1, X) for M1, X in zip(M1h, Xh)]
        Xh = [_dot(X, 2.0 * eye2 - Y) for X, Y in zip(Xh, Yh)]
    T21s = [_dot(Ls[t][h2:, :h2], Xh[t]) for t in range(nh)]
    X21s = [-_dot(Xh[nh + t], T21s[t]) for t in range(nh)]
    z2 = jnp.zeros((h2, h2), jnp.float32)
    Xs = [jnp.concatenate(
        [jnp.concatenate([Xh[t], z2], axis=1),
         jnp.concatenate([X21s[t], Xh[nh + t]], axis=1)], axis=0)
        for t in range(nh)]
    Mlows = [jnp.where(lower, _dot_t(W, K), 0.0) for W, K in zip(Ws, Ks)]
    Gs = [jnp.where(lower_eq, _dot_t(Q, W), 0.0) * bv
          for Q, W, bv in zip(Qs, Ws, bvs)]
    GTs = [_dot(G, X) for G, X in zip(Gs, Xs)]
    Cms = [_dot(X, Mlow) for X, Mlow in zip(Xs, Mlows)]
    QKs = [_dot_t(Q, K) for Q, K in zip(Qs, Ks)]
    BWs = [jnp.transpose(bv) * W for bv, W in zip(bvs, Ws)]
    XWs = [_dot(X, W) for X, W in zip(Xs, Ws)]
    for t in range(nh):
        sii_ref[t, 0] = (QKs[t] - _dot(GTs[t], Mlows[t])) * scale
        qt_ref[t] = (Qs[t] - _dot(GTs[t], Ws[t])) * scale
        kt_ref[t] = Ks[t] - _dot_tt(Cms[t], BWs[t])
        p_ref[t, 0] = eye_h - _dot_tt(BWs[t], XWs[t])


def _flash_kernel(qt_ref, sii_ref, kt_ref, v_ref, pm_ref, logf_ref,
                  o_ref, fc_ref, fcg_ref, khat_ref, *, c, hd, nc, nb):
    ii = pl.program_id(1)

    @pl.when(ii == 0)
    def _():
        khat_ref[...] = jnp.zeros_like(khat_ref)
        fcg_ref[...] = jnp.full_like(fcg_ref, 1e30)
        crows = jax.lax.broadcasted_iota(jnp.int32, (c, c), 0)
        ccols = jax.lax.broadcasted_iota(jnp.int32, (c, c), 1)
        u_inc = jnp.where(crows <= ccols, 1.0, 0.0).astype(jnp.float32)
        nrows = jax.lax.broadcasted_iota(jnp.int32, (nc, nc), 0)
        ncols = jax.lax.broadcasted_iota(jnp.int32, (nc, nc), 1)
        l_strict = jnp.where(nrows > ncols, 1.0, 0.0).astype(jnp.float32)
        for t in range(nb):
            row_cum = _dot(logf_ref[0, t], u_inc)
            offs = _dot(l_strict, row_cum[:, c - 1:c])
            fc_ref[t] = row_cum + offs

    rows = jax.lax.broadcasted_iota(jnp.int32, (c, c), 0)
    cols = jax.lax.broadcasted_iota(jnp.int32, (c, c), 1)
    tri = rows >= cols

    fq_rows = [fc_ref[t, pl.ds(ii, 1), :] for t in range(nb)]
    fq_cols = [jnp.transpose(fq) for fq in fq_rows]
    Ss = [_dot_t(qt_ref[t], khat_ref[t]) for t in range(nb)]
    lbds = [jnp.where(tri, sii_ref[t, 0] + fq_cols[t] - fq_rows[t], -1e30)
            for t in range(nb)]
    ms = [lbd.max(axis=1, keepdims=True) for lbd in lbds]
    all_blocks = []
    for t in range(nb):
        blocks = []
        m = ms[t]
        for r in range(nc):
            lb = Ss[t][:, r * c:(r + 1) * c] + fq_cols[t] - fcg_ref[t, r:r + 1, :]
            blocks.append(lb)
            m = jnp.maximum(m, lb.max(axis=1, keepdims=True))
        ms[t] = m
        all_blocks.append(blocks)
    Eds = [jnp.exp(lbds[t] - ms[t]) for t in range(nb)]
    Es = [jnp.concatenate(
        [jnp.exp(bk - ms[t]) for bk in all_blocks[t]], axis=1)
        for t in range(nb)]
    v_iis = [v_ref[t, pl.ds(ii * c, c), :] for t in range(nb)]
    pvls = [_dot(Es[t], v_ref[t]) + _dot(Eds[t], v_iis[t]) for t in range(nb)]
    for t in range(nb):
        o_ref[t] = pvls[t][:, :hd] / pvls[t][:, hd:hd + 1]
    for t in range(nb):
        khat_ref[t] = _dot_t(khat_ref[t], pm_ref[t, 0])
    for t in range(nb):
        khat_ref[t, pl.ds(ii * c, c), :] = kt_ref[t]
    for t in range(nb):
        fcg_ref[t, pl.ds(ii, 1), :] = fc_ref[t, pl.ds(ii, 1), :]


def _outproj_kernel(o_ref, wo_ref, out_ref, *, h):
    om = jnp.concatenate([o_ref[hh] for hh in range(h)], axis=1)
    out_ref[0] = _dot(om, wo_ref[...])


def kernel(x, Wq, Wk, Wv, Wo, Ww1, Ww2, Wb, Wf, delta):
    b, n, d = x.shape
    h = delta.shape[0]
    hd = d // h
    nc = n // C
    bh = b * h
    scale = hd ** -0.5
    newton_iters = max(1, (C - 1).bit_length() - 1)

    wqkv = jnp.concatenate([Wq, Wk, Wv], axis=1)
    delta2 = delta.reshape(1, h)

    f32 = jnp.float32
    q, k, v, w, beta, logf = pl.pallas_call(
        functools.partial(_proj_kernel, h=h, hd=hd),
        grid=(b, nc),
        in_specs=[
            pl.BlockSpec((1, C, d), lambda bi, ci: (bi, ci, 0)),
            pl.BlockSpec((d, 3 * d), lambda bi, ci: (0, 0)),
            pl.BlockSpec((d, hd), lambda bi, ci: (0, 0)),
            pl.BlockSpec((hd, d), lambda bi, ci: (0, 0)),
            pl.BlockSpec((d, h), lambda bi, ci: (0, 0)),
            pl.BlockSpec((d, h), lambda bi, ci: (0, 0)),
            pl.BlockSpec((1, h), lambda bi, ci: (0, 0)),
        ],
        out_specs=[
            pl.BlockSpec((h, C, hd), lambda bi, ci: (bi, ci, 0)),
            pl.BlockSpec((h, C, hd), lambda bi, ci: (bi, ci, 0)),
            pl.BlockSpec((h, C, hd + 1), lambda bi, ci: (bi, ci, 0)),
            pl.BlockSpec((h, C, hd), lambda bi, ci: (bi, ci, 0)),
            pl.BlockSpec((1, h, 1, C), lambda bi, ci: (bi, 0, 0, ci)),
            pl.BlockSpec((1, h, 1, C), lambda bi, ci: (bi, 0, 0, ci)),
        ],
        out_shape=[
            jax.ShapeDtypeStruct((bh, n, hd), f32),
            jax.ShapeDtypeStruct((bh, n, hd), f32),
            jax.ShapeDtypeStruct((bh, n, hd + 1), f32),
            jax.ShapeDtypeStruct((bh, n, hd), f32),
            jax.ShapeDtypeStruct((b, h, 1, n), f32),
            jax.ShapeDtypeStruct((b, h, 1, n), f32),
        ],
        compiler_params=pltpu.CompilerParams(
            dimension_semantics=("parallel", "arbitrary"),
            vmem_limit_bytes=52 * 1024 * 1024,
        ),
        name="path_proj",
    )(x, wqkv, Ww1, Ww2, Wb, Wf, delta2)

    NH = 8
    qt, kt, sii, pm = pl.pallas_call(
        functools.partial(_ut_kernel, c=C, hd=hd, newton_iters=newton_iters,
                          nh=NH, scale=scale),
        grid=(bh // NH, nc),
        in_specs=[
            pl.BlockSpec((NH, C, hd), lambda g, ci: (g, ci, 0)),
            pl.BlockSpec((NH, C, hd), lambda g, ci: (g, ci, 0)),
            pl.BlockSpec((NH, C, hd), lambda g, ci: (g, ci, 0)),
            pl.BlockSpec((1, NH, 1, C),
                         lambda g, ci: (g // (h // NH), g % (h // NH), 0, ci)),
        ],
        out_specs=[
            pl.BlockSpec((NH, C, hd), lambda g, ci: (g, ci, 0)),
            pl.BlockSpec((NH, C, hd), lambda g, ci: (g, ci, 0)),
            pl.BlockSpec((NH, 1, C, C), lambda g, ci: (g, ci, 0, 0)),
            pl.BlockSpec((NH, 1, hd, hd), lambda g, ci: (g, ci, 0, 0)),
        ],
        out_shape=[
            jax.ShapeDtypeStruct((bh, n, hd), f32),
            jax.ShapeDtypeStruct((bh, n, hd), f32),
            jax.ShapeDtypeStruct((bh, nc, C, C), f32),
            jax.ShapeDtypeStruct((bh, nc, hd, hd), f32),
        ],
        compiler_params=pltpu.CompilerParams(
            dimension_semantics=("parallel", "arbitrary"),
        ),
        name="path_ut",
    )(q, k, w, beta)

    NB = 16
    o = pl.pallas_call(
        functools.partial(_flash_kernel, c=C, hd=hd, nc=nc, nb=NB),
        grid=(bh // NB, nc),
        in_specs=[
            pl.BlockSpec((NB, C, hd), lambda g, ii: (g, ii, 0)),
            pl.BlockSpec((NB, 1, C, C), lambda g, ii: (g, ii, 0, 0)),
            pl.BlockSpec((NB, C, hd), lambda g, ii: (g, ii, 0)),
            pl.BlockSpec((NB, n, hd + 1), lambda g, ii: (g, 0, 0)),
            pl.BlockSpec((NB, 1, hd, hd), lambda g, ii: (g, ii, 0, 0)),
            pl.BlockSpec((1, NB, nc, C),
                         lambda g, ii: (g // (h // NB), g % (h // NB), 0, 0)),
        ],
        out_specs=pl.BlockSpec((NB, C, hd), lambda g, ii: (g, ii, 0)),
        out_shape=jax.ShapeDtypeStruct((bh, n, hd), f32),
        scratch_shapes=[
            pltpu.VMEM((NB, nc, C), f32),
            pltpu.VMEM((NB, nc, C), f32),
            pltpu.VMEM((NB, n, hd), f32),
        ],
        compiler_params=pltpu.CompilerParams(
            dimension_semantics=("parallel", "arbitrary"),
        ),
        name="path_flash",
    )(qt, sii, kt, v, pm, logf.reshape(b, h, nc, C))

    RB = 256
    out = pl.pallas_call(
        functools.partial(_outproj_kernel, h=h),
        grid=(b, n // RB),
        in_specs=[
            pl.BlockSpec((h, RB, hd), lambda bi, ri: (bi, ri, 0)),
            pl.BlockSpec((d, d), lambda bi, ri: (0, 0)),
        ],
        out_specs=pl.BlockSpec((1, RB, d), lambda bi, ri: (bi, ri, 0)),
        out_shape=jax.ShapeDtypeStruct((b, n, d), f32),
        compiler_params=pltpu.CompilerParams(
            dimension_semantics=("parallel", "arbitrary"),
        ),
        name="path_outproj",
    )(o, Wo)
    return out
